# initial kernel scaffold (unmeasured)
import jax
import jax.numpy as jnp
from jax import lax
from jax.experimental import pallas as pl
from jax.experimental.pallas import tpu as pltpu

M = 4096
N = 8192
K = 4096
HALF = M // 2


def _matmul_body(x_ref, dy_ref, o_ref, acc_ref):
    @pl.when(pl.program_id(2) == 0)
    def _():
        acc_ref[...] = jnp.zeros_like(acc_ref)

    xb = x_ref[...].astype(jnp.bfloat16)
    db = dy_ref[...].astype(jnp.bfloat16)
    acc_ref[...] += lax.dot_general(
        xb, db, (((0,), (0,)), ((), ())), preferred_element_type=jnp.float32
    )

    @pl.when(pl.program_id(2) == pl.num_programs(2) - 1)
    def _():
        o_ref[...] = acc_ref[...].astype(jnp.bfloat16)


def _matmul(x_ro, dy):
    bm, bn, bk = 512, 1024, 512
    grid = (M // bm, N // bn, K // bk)
    return pl.pallas_call(
        _matmul_body,
        grid=grid,
        in_specs=[
            pl.BlockSpec((bk, bm), lambda i, j, k: (k, i)),
            pl.BlockSpec((bk, bn), lambda i, j, k: (k, j)),
        ],
        out_specs=pl.BlockSpec((bm, bn), lambda i, j, k: (i, j)),
        out_shape=jax.ShapeDtypeStruct((M, N), jnp.bfloat16),
        scratch_shapes=[pltpu.VMEM((bm, bn), jnp.float32)],
        compiler_params=pltpu.CompilerParams(
            dimension_semantics=("parallel", "parallel", "arbitrary"),
        ),
    )(x_ro, dy)


def _exchange_body(p_ref, out_ref, send_sem, recv_sem):
    g = lax.axis_index("x")
    my_y = lax.axis_index("y")
    my_z = lax.axis_index("z")
    rdma = pltpu.make_async_remote_copy(
        src_ref=p_ref.at[pl.ds(HALF, HALF), :],
        dst_ref=out_ref,
        send_sem=send_sem,
        recv_sem=recv_sem,
        device_id=(1 - g, my_y, my_z),
        device_id_type=pl.DeviceIdType.MESH,
    )
    rdma.start()
    rdma.wait()


def _exchange(p):
    return pl.pallas_call(
        _exchange_body,
        in_specs=[pl.BlockSpec(memory_space=pltpu.ANY)],
        out_specs=pl.BlockSpec(memory_space=pltpu.ANY),
        out_shape=jax.ShapeDtypeStruct((HALF, N), jnp.bfloat16),
        scratch_shapes=[
            pltpu.SemaphoreType.DMA,
            pltpu.SemaphoreType.DMA,
        ],
        compiler_params=pltpu.CompilerParams(collective_id=0),
    )(p)


def _add_body(a_ref, b_ref, o_ref):
    o_ref[...] = a_ref[...] + b_ref[...]


def _add(p, recv):
    bm, bn = 512, 2048
    grid = (HALF // bm, N // bn)
    return pl.pallas_call(
        _add_body,
        grid=grid,
        in_specs=[
            pl.BlockSpec((bm, bn), lambda i, j: (i, j)),
            pl.BlockSpec((bm, bn), lambda i, j: (i, j)),
        ],
        out_specs=pl.BlockSpec((bm, bn), lambda i, j: (i, j)),
        out_shape=jax.ShapeDtypeStruct((HALF, N), jnp.bfloat16),
    )(p, recv)


def kernel(x, dy):
    g = lax.axis_index("x")
    x_me = lax.dynamic_slice(x, (0, g * HALF), (K, HALF))
    x_other = lax.dynamic_slice(x, (0, (1 - g) * HALF), (K, HALF))
    x_ro = jnp.concatenate([x_me, x_other], axis=1)

    p = _matmul(x_ro, dy)
    recv = _exchange(p)
    return _add(p, recv)


# baseline (device time: 1251842 ns/iter reference)
import jax
import jax.numpy as jnp
from jax import lax
from jax.experimental import pallas as pl
from jax.experimental.pallas import tpu as pltpu

M = 4096
N = 8192
K = 4096
HALF = M // 2


def _matmul_body(x_ref, dy_ref, o_ref, acc_ref):
    @pl.when(pl.program_id(2) == 0)
    def _():
        acc_ref[...] = jnp.zeros_like(acc_ref)

    xb = x_ref[...].astype(jnp.bfloat16)
    db = dy_ref[...].astype(jnp.bfloat16)
    acc_ref[...] += lax.dot_general(
        xb, db, (((0,), (0,)), ((), ())), preferred_element_type=jnp.float32
    )

    @pl.when(pl.program_id(2) == pl.num_programs(2) - 1)
    def _():
        o_ref[...] = acc_ref[...].astype(jnp.bfloat16)


def _matmul(x_ro, dy):
    bm, bn, bk = 512, 1024, 512
    grid = (M // bm, N // bn, K // bk)
    return pl.pallas_call(
        _matmul_body,
        grid=grid,
        in_specs=[
            pl.BlockSpec((bk, bm), lambda i, j, k: (k, i)),
            pl.BlockSpec((bk, bn), lambda i, j, k: (k, j)),
        ],
        out_specs=pl.BlockSpec((bm, bn), lambda i, j, k: (i, j)),
        out_shape=jax.ShapeDtypeStruct((M, N), jnp.bfloat16),
        scratch_shapes=[pltpu.VMEM((bm, bn), jnp.float32)],
        compiler_params=pltpu.CompilerParams(
            dimension_semantics=("parallel", "parallel", "arbitrary"),
        ),
    )(x_ro, dy)


def _exchange_body(p_ref, out_ref, send_sem, recv_sem):
    g = lax.axis_index("x")
    my_y = lax.axis_index("y")
    my_z = lax.axis_index("z")
    rdma = pltpu.make_async_remote_copy(
        src_ref=p_ref.at[pl.ds(HALF, HALF), :],
        dst_ref=out_ref,
        send_sem=send_sem,
        recv_sem=recv_sem,
        device_id=(1 - g, my_y, my_z),
        device_id_type=pl.DeviceIdType.MESH,
    )
    rdma.start()
    rdma.wait()


def _exchange(p):
    return pl.pallas_call(
        _exchange_body,
        in_specs=[pl.BlockSpec(memory_space=pltpu.MemorySpace.HBM)],
        out_specs=pl.BlockSpec(memory_space=pltpu.MemorySpace.HBM),
        out_shape=jax.ShapeDtypeStruct((HALF, N), jnp.bfloat16),
        scratch_shapes=[
            pltpu.SemaphoreType.DMA,
            pltpu.SemaphoreType.DMA,
        ],
    )(p)


def _add_body(a_ref, b_ref, o_ref):
    o_ref[...] = a_ref[...] + b_ref[...]


def _add(p, recv):
    bm, bn = 512, 2048
    grid = (HALF // bm, N // bn)
    return pl.pallas_call(
        _add_body,
        grid=grid,
        in_specs=[
            pl.BlockSpec((bm, bn), lambda i, j: (i, j)),
            pl.BlockSpec((bm, bn), lambda i, j: (i, j)),
        ],
        out_specs=pl.BlockSpec((bm, bn), lambda i, j: (i, j)),
        out_shape=jax.ShapeDtypeStruct((HALF, N), jnp.bfloat16),
    )(p, recv)


def kernel(x, dy):
    g = lax.axis_index("x")
    x_me = lax.dynamic_slice(x, (0, g * HALF), (K, HALF))
    x_other = lax.dynamic_slice(x, (0, (1 - g) * HALF), (K, HALF))
    x_ro = jnp.concatenate([x_me, x_other], axis=1)

    p = _matmul(x_ro, dy)
    recv = _exchange(p)
    return _add(p, recv)


# device time: 621012 ns/iter; 2.0158x vs baseline; 2.0158x over previous
import jax
import jax.numpy as jnp
from jax import lax
from jax.experimental import pallas as pl
from jax.experimental.pallas import tpu as pltpu

M = 4096
N = 8192
K = 4096
HALF = M // 2
NSTRIP = 8
SW = N // NSTRIP


def _ring_pos(y, z):
    return 2 * z + (y + z) % 2


def _pos_to_yz(p):
    z = p // 2
    y = (p % 2 + z) % 2
    return y, z


def _matmul_body(x_ref, dy_ref, o_ref, acc_ref):
    @pl.when(pl.program_id(2) == 0)
    def _():
        acc_ref[...] = jnp.zeros_like(acc_ref)

    xb = x_ref[...].astype(jnp.bfloat16)
    db = dy_ref[...].astype(jnp.bfloat16)
    acc_ref[...] += lax.dot_general(
        xb, db, (((0,), (0,)), ((), ())), preferred_element_type=jnp.float32
    )

    @pl.when(pl.program_id(2) == pl.num_programs(2) - 1)
    def _():
        o_ref[...] = acc_ref[...].astype(jnp.bfloat16)


def _matmul(x_ro, dy_strip):
    bm, bn, bk = 512, 512, 1024
    grid = (M // bm, SW // bn, K // bk)
    return pl.pallas_call(
        _matmul_body,
        grid=grid,
        in_specs=[
            pl.BlockSpec((bk, bm), lambda i, j, k: (k, i)),
            pl.BlockSpec((bk, bn), lambda i, j, k: (k, j)),
        ],
        out_specs=pl.BlockSpec((bm, bn), lambda i, j, k: (i, j)),
        out_shape=jax.ShapeDtypeStruct((M, SW), jnp.bfloat16),
        scratch_shapes=[pltpu.VMEM((bm, bn), jnp.float32)],
        compiler_params=pltpu.CompilerParams(
            dimension_semantics=("parallel", "parallel", "arbitrary"),
        ),
    )(x_ro, dy_strip)


def _comm_body(
    p_ref, out_ref, xbuf, vtmp, vstrip, copy_sem, xsend, xrecv, send_sems, recv_sems
):
    g = lax.axis_index("x")
    y = lax.axis_index("y")
    z = lax.axis_index("z")
    p = _ring_pos(y, z)
    pr = (p + 1) % NSTRIP
    ry, rz = _pos_to_yz(pr)

    x_rdma = pltpu.make_async_remote_copy(
        src_ref=p_ref.at[pl.ds(HALF, HALF), :],
        dst_ref=xbuf,
        send_sem=xsend,
        recv_sem=xrecv,
        device_id=(1 - g, y, z),
        device_id_type=pl.DeviceIdType.MESH,
    )
    x_rdma.start()

    cp_in = pltpu.make_async_copy(p_ref.at[pl.ds(0, HALF), :], vtmp, copy_sem)
    cp_in.start()
    cp_in.wait()
    x_rdma.wait()

    vstrip[...] = vtmp[...] + xbuf[...]
    cp_out = pltpu.make_async_copy(
        vstrip, out_ref.at[:, pl.ds(p * SW, SW)], copy_sem
    )
    cp_out.start()
    cp_out.wait()

    for h in range(NSTRIP - 1):
        o_send = (p - h) % NSTRIP
        rdma = pltpu.make_async_remote_copy(
            src_ref=out_ref.at[:, pl.ds(o_send * SW, SW)],
            dst_ref=out_ref.at[:, pl.ds(o_send * SW, SW)],
            send_sem=send_sems.at[h],
            recv_sem=recv_sems.at[h],
            device_id=(g, ry, rz),
            device_id_type=pl.DeviceIdType.MESH,
        )
        rdma.start()
        rdma.wait()


def _comm(p_strip):
    return pl.pallas_call(
        _comm_body,
        in_specs=[pl.BlockSpec(memory_space=pltpu.MemorySpace.HBM)],
        out_specs=pl.BlockSpec(memory_space=pltpu.MemorySpace.HBM),
        out_shape=jax.ShapeDtypeStruct((HALF, N), jnp.bfloat16),
        scratch_shapes=[
            pltpu.VMEM((HALF, SW), jnp.bfloat16),
            pltpu.VMEM((HALF, SW), jnp.bfloat16),
            pltpu.VMEM((HALF, SW), jnp.bfloat16),
            pltpu.SemaphoreType.DMA,
            pltpu.SemaphoreType.DMA,
            pltpu.SemaphoreType.DMA,
            pltpu.SemaphoreType.DMA((NSTRIP - 1,)),
            pltpu.SemaphoreType.DMA((NSTRIP - 1,)),
        ],
    )(p_strip)


def kernel(x, dy):
    g = lax.axis_index("x")
    y = lax.axis_index("y")
    z = lax.axis_index("z")
    p = _ring_pos(y, z)

    x_me = lax.dynamic_slice(x, (0, g * HALF), (K, HALF))
    x_other = lax.dynamic_slice(x, (0, (1 - g) * HALF), (K, HALF))
    x_ro = jnp.concatenate([x_me, x_other], axis=1)

    dy_strip = lax.dynamic_slice(dy, (0, p * SW), (K, SW))

    p_strip = _matmul(x_ro, dy_strip)
    return _comm(p_strip)


# device time: 361085 ns/iter; 3.4669x vs baseline; 1.7198x over previous
import jax
import jax.numpy as jnp
from jax import lax
from jax.experimental import pallas as pl
from jax.experimental.pallas import tpu as pltpu

M = 4096
N = 8192
K = 4096
HALF = M // 2
NDEV = 8
SW = N // NDEV
HW = SW // 2


def _h1_pos(y, z):
    return 2 * z + (y + z) % 2


def _h1_yz(p):
    z = p // 2
    y = (p % 2 + z) % 2
    return y, z


def _h2_pos(y, z):
    q0 = z + 2 * (z // 2)
    q1 = jnp.where(z == 0, 7, jnp.where(z == 3, 6, z + 1))
    return jnp.where(y == 0, q0, q1)


def _h2_yz(q):
    y = (q // 2) % 2
    z = ((q + 1) // 2) % 4
    return y, z


def _matmul_body(s_ref, x_ref, dy_ref, o_ref, acc_ref):
    @pl.when(pl.program_id(2) == 0)
    def _():
        acc_ref[...] = jnp.zeros_like(acc_ref)

    xb = x_ref[...].astype(jnp.bfloat16)
    db = dy_ref[...].astype(jnp.bfloat16)
    acc_ref[...] += lax.dot_general(
        xb, db, (((0,), (0,)), ((), ())), preferred_element_type=jnp.float32
    )

    @pl.when(pl.program_id(2) == pl.num_programs(2) - 1)
    def _():
        o_ref[...] = acc_ref[...].astype(jnp.bfloat16)


_BM, _BN, _BK = 512, 512, 1024


def _matmul(scalars, x, dy):
    grid = (M // _BM, SW // _BN, K // _BK)
    grid_spec = pltpu.PrefetchScalarGridSpec(
        num_scalar_prefetch=1,
        grid=grid,
        in_specs=[
            pl.BlockSpec(
                (_BK, _BM),
                lambda i, j, k, s: (k, (i + s[0] * (HALF // _BM)) % (M // _BM)),
            ),
            pl.BlockSpec(
                (_BK, _BN), lambda i, j, k, s: (k, j + s[1] * (SW // _BN))
            ),
        ],
        out_specs=pl.BlockSpec((_BM, _BN), lambda i, j, k, s: (i, j)),
        scratch_shapes=[pltpu.VMEM((_BM, _BN), jnp.float32)],
    )
    return pl.pallas_call(
        _matmul_body,
        grid_spec=grid_spec,
        out_shape=jax.ShapeDtypeStruct((M, SW), jnp.bfloat16),
        compiler_params=pltpu.CompilerParams(
            dimension_semantics=("parallel", "parallel", "arbitrary"),
        ),
    )(scalars, x, dy)


def _comm_body(
    p_ref, out_ref, xbuf, vtmp, vstrip, copy_sem,
    xsend, xrecv, a_send, a_recv, b_send, b_recv
):
    g = lax.axis_index("x")
    y = lax.axis_index("y")
    z = lax.axis_index("z")
    p = _h1_pos(y, z)
    q = _h2_pos(y, z)
    ay, az = _h1_yz((p + 1) % NDEV)
    by, bz = _h2_yz((q + 1) % NDEV)

    x_rdma = pltpu.make_async_remote_copy(
        src_ref=p_ref.at[pl.ds(HALF, HALF), :],
        dst_ref=xbuf,
        send_sem=xsend,
        recv_sem=xrecv,
        device_id=(1 - g, y, z),
        device_id_type=pl.DeviceIdType.MESH,
    )
    x_rdma.start()

    cp_in = pltpu.make_async_copy(p_ref.at[pl.ds(0, HALF), :], vtmp, copy_sem)
    cp_in.start()
    cp_in.wait()
    x_rdma.wait()

    vstrip[...] = vtmp[...] + xbuf[...]
    cp_out = pltpu.make_async_copy(
        vstrip, out_ref.at[:, pl.ds(p * SW, SW)], copy_sem
    )
    cp_out.start()
    cp_out.wait()

    for h in range(NDEV - 1):
        oa = (p - h) % NDEV
        qb = (q - h) % NDEV
        oby, obz = _h2_yz(qb)
        ob = _h1_pos(oby, obz)

        rdma_a = pltpu.make_async_remote_copy(
            src_ref=out_ref.at[:, pl.ds(oa * SW, HW)],
            dst_ref=out_ref.at[:, pl.ds(oa * SW, HW)],
            send_sem=a_send.at[h],
            recv_sem=a_recv.at[h],
            device_id=(g, ay, az),
            device_id_type=pl.DeviceIdType.MESH,
        )
        rdma_b = pltpu.make_async_remote_copy(
            src_ref=out_ref.at[:, pl.ds(ob * SW + HW, HW)],
            dst_ref=out_ref.at[:, pl.ds(ob * SW + HW, HW)],
            send_sem=b_send.at[h],
            recv_sem=b_recv.at[h],
            device_id=(g, by, bz),
            device_id_type=pl.DeviceIdType.MESH,
        )
        rdma_a.start()
        rdma_b.start()
        rdma_a.wait()
        rdma_b.wait()


def _comm(p_strip):
    return pl.pallas_call(
        _comm_body,
        in_specs=[pl.BlockSpec(memory_space=pltpu.MemorySpace.HBM)],
        out_specs=pl.BlockSpec(memory_space=pltpu.MemorySpace.HBM),
        out_shape=jax.ShapeDtypeStruct((HALF, N), jnp.bfloat16),
        scratch_shapes=[
            pltpu.VMEM((HALF, SW), jnp.bfloat16),
            pltpu.VMEM((HALF, SW), jnp.bfloat16),
            pltpu.VMEM((HALF, SW), jnp.bfloat16),
            pltpu.SemaphoreType.DMA,
            pltpu.SemaphoreType.DMA,
            pltpu.SemaphoreType.DMA,
            pltpu.SemaphoreType.DMA((NDEV - 1,)),
            pltpu.SemaphoreType.DMA((NDEV - 1,)),
            pltpu.SemaphoreType.DMA((NDEV - 1,)),
            pltpu.SemaphoreType.DMA((NDEV - 1,)),
        ],
    )(p_strip)


def kernel(x, dy):
    g = lax.axis_index("x")
    y = lax.axis_index("y")
    z = lax.axis_index("z")
    p = _h1_pos(y, z)

    scalars = jnp.stack([g, p]).astype(jnp.int32)
    p_strip = _matmul(scalars, x, dy)
    return _comm(p_strip)


# device time: 288206 ns/iter; 4.3436x vs baseline; 1.2529x over previous
import jax
import jax.numpy as jnp
from jax import lax
from jax.experimental import pallas as pl
from jax.experimental.pallas import tpu as pltpu

M = 4096
N = 8192
K = 4096
HALF = M // 2
NDEV = 8
SW = N // NDEV
HW = SW // 2

_BM = 512
_BK = 1024
_NB = HALF // _BM


def _h1_pos(y, z):
    return 2 * z + (y + z) % 2


def _h1_yz(p):
    z = p // 2
    y = (p % 2 + z) % 2
    return y, z


def _h2_pos(y, z):
    q0 = z + 2 * (z // 2)
    q1 = jnp.where(z == 0, 7, jnp.where(z == 3, 6, z + 1))
    return jnp.where(y == 0, q0, q1)


def _h2_yz(q):
    y = (q // 2) % 2
    z = ((q + 1) // 2) % 4
    return y, z


def _mm_body(s_ref, x_ref, dy_ref, o_ref, acc, sblk, rbuf, ssem, rsem):
    i = pl.program_id(0)
    k = pl.program_id(1)
    g = s_ref[0]
    y = lax.axis_index("y")
    z = lax.axis_index("z")

    @pl.when(k == 0)
    def _():
        acc[...] = jnp.zeros_like(acc)

    xb = x_ref[...].astype(jnp.bfloat16)
    db = dy_ref[...].astype(jnp.bfloat16)
    acc[...] += lax.dot_general(
        xb, db, (((0,), (0,)), ((), ())), preferred_element_type=jnp.float32
    )

    @pl.when(k == K // _BK - 1)
    def _():
        for b in range(_NB):
            @pl.when(i == b)
            def _():
                sblk[b] = acc[...].astype(jnp.bfloat16)
                pltpu.make_async_remote_copy(
                    src_ref=sblk.at[b],
                    dst_ref=rbuf.at[b],
                    send_sem=ssem.at[b],
                    recv_sem=rsem.at[b],
                    device_id=(1 - g, y, z),
                    device_id_type=pl.DeviceIdType.MESH,
                ).start()

            @pl.when(i == _NB + b)
            def _():
                pltpu.make_async_remote_copy(
                    src_ref=sblk.at[b],
                    dst_ref=rbuf.at[b],
                    send_sem=ssem.at[b],
                    recv_sem=rsem.at[b],
                    device_id=(1 - g, y, z),
                    device_id_type=pl.DeviceIdType.MESH,
                ).wait_recv()
                o_ref[...] = (acc[...] + rbuf[b].astype(jnp.float32)).astype(
                    jnp.bfloat16
                )

        @pl.when(i == 2 * _NB - 1)
        def _():
            for b in range(_NB):
                pltpu.make_async_remote_copy(
                    src_ref=sblk.at[b],
                    dst_ref=rbuf.at[b],
                    send_sem=ssem.at[b],
                    recv_sem=rsem.at[b],
                    device_id=(1 - g, y, z),
                    device_id_type=pl.DeviceIdType.MESH,
                ).wait_send()


def _matmul_reduce(scalars, x, dy):
    grid = (2 * _NB, K // _BK)
    grid_spec = pltpu.PrefetchScalarGridSpec(
        num_scalar_prefetch=1,
        grid=grid,
        in_specs=[
            pl.BlockSpec(
                (_BK, _BM),
                lambda i, k, s: (
                    k,
                    jnp.where(
                        i < _NB,
                        (1 - s[0]) * _NB + i,
                        s[0] * _NB + (i - _NB),
                    ),
                ),
            ),
            pl.BlockSpec((_BK, SW), lambda i, k, s: (k, s[1])),
        ],
        out_specs=pl.BlockSpec(
            (_BM, SW),
            lambda i, k, s: (jnp.where(i < _NB, _NB, i - _NB), 0),
        ),
        scratch_shapes=[
            pltpu.VMEM((_BM, SW), jnp.float32),
            pltpu.VMEM((_NB, _BM, SW), jnp.bfloat16),
            pltpu.VMEM((_NB, _BM, SW), jnp.bfloat16),
            pltpu.SemaphoreType.DMA((_NB,)),
            pltpu.SemaphoreType.DMA((_NB,)),
        ],
    )
    return pl.pallas_call(
        _mm_body,
        grid_spec=grid_spec,
        out_shape=jax.ShapeDtypeStruct((HALF + _BM, SW), jnp.bfloat16),
        compiler_params=pltpu.CompilerParams(
            dimension_semantics=("arbitrary", "arbitrary"),
        ),
    )(scalars, x, dy)


def _comm_body(f_ref, out_ref, copy_sem, a_send, a_recv, b_send, b_recv):
    g = lax.axis_index("x")
    y = lax.axis_index("y")
    z = lax.axis_index("z")
    p = _h1_pos(y, z)
    q = _h2_pos(y, z)
    ay, az = _h1_yz((p + 1) % NDEV)
    by, bz = _h2_yz((q + 1) % NDEV)

    cp = pltpu.make_async_copy(
        f_ref.at[pl.ds(0, HALF), :], out_ref.at[:, pl.ds(p * SW, SW)], copy_sem
    )
    cp.start()
    cp.wait()

    for h in range(NDEV - 1):
        oa = (p - h) % NDEV
        qb = (q - h) % NDEV
        oby, obz = _h2_yz(qb)
        ob = _h1_pos(oby, obz)

        rdma_a = pltpu.make_async_remote_copy(
            src_ref=out_ref.at[:, pl.ds(oa * SW, HW)],
            dst_ref=out_ref.at[:, pl.ds(oa * SW, HW)],
            send_sem=a_send.at[h],
            recv_sem=a_recv.at[h],
            device_id=(g, ay, az),
            device_id_type=pl.DeviceIdType.MESH,
        )
        rdma_b = pltpu.make_async_remote_copy(
            src_ref=out_ref.at[:, pl.ds(ob * SW + HW, HW)],
            dst_ref=out_ref.at[:, pl.ds(ob * SW + HW, HW)],
            send_sem=b_send.at[h],
            recv_sem=b_recv.at[h],
            device_id=(g, by, bz),
            device_id_type=pl.DeviceIdType.MESH,
        )
        rdma_a.start()
        rdma_b.start()
        rdma_a.wait()
        rdma_b.wait()


def _comm(f_strip):
    return pl.pallas_call(
        _comm_body,
        in_specs=[pl.BlockSpec(memory_space=pltpu.MemorySpace.HBM)],
        out_specs=pl.BlockSpec(memory_space=pltpu.MemorySpace.HBM),
        out_shape=jax.ShapeDtypeStruct((HALF, N), jnp.bfloat16),
        scratch_shapes=[
            pltpu.SemaphoreType.DMA,
            pltpu.SemaphoreType.DMA((NDEV - 1,)),
            pltpu.SemaphoreType.DMA((NDEV - 1,)),
            pltpu.SemaphoreType.DMA((NDEV - 1,)),
            pltpu.SemaphoreType.DMA((NDEV - 1,)),
        ],
    )(f_strip)


def kernel(x, dy):
    g = lax.axis_index("x")
    y = lax.axis_index("y")
    z = lax.axis_index("z")
    p = _h1_pos(y, z)

    scalars = jnp.stack([g, p]).astype(jnp.int32)
    f_strip = _matmul_reduce(scalars, x, dy)
    return _comm(f_strip)


# device time: 275533 ns/iter; 4.5433x vs baseline; 1.0460x over previous
import jax
import jax.numpy as jnp
from jax import lax
from jax.experimental import pallas as pl
from jax.experimental.pallas import tpu as pltpu

M = 4096
N = 8192
K = 4096
HALF = M // 2
NDEV = 8
SW = N // NDEV
HW = SW // 2

_BM = 1024
_BK = 1024
_NB = HALF // _BM


def _h1_pos(y, z):
    return 2 * z + (y + z) % 2


def _h1_yz(p):
    z = p // 2
    y = (p % 2 + z) % 2
    return y, z


def _h2_pos(y, z):
    q0 = z + 2 * (z // 2)
    q1 = jnp.where(z == 0, 7, jnp.where(z == 3, 6, z + 1))
    return jnp.where(y == 0, q0, q1)


def _h2_yz(q):
    y = (q // 2) % 2
    z = ((q + 1) // 2) % 4
    return y, z


def _mm_body(s_ref, x_ref, dy_ref, o_ref, acc, sblk, rbuf, ssem, rsem):
    i = pl.program_id(0)
    k = pl.program_id(1)
    g = s_ref[0]
    y = lax.axis_index("y")
    z = lax.axis_index("z")

    @pl.when(k == 0)
    def _():
        acc[...] = jnp.zeros_like(acc)

    xb = x_ref[...].astype(jnp.bfloat16)
    db = dy_ref[...].astype(jnp.bfloat16)
    acc[...] += lax.dot_general(
        xb, db, (((0,), (0,)), ((), ())), preferred_element_type=jnp.float32
    )

    @pl.when(k == K // _BK - 1)
    def _():
        for b in range(_NB):
            @pl.when(i == b)
            def _():
                sblk[b] = acc[...].astype(jnp.bfloat16)
                pltpu.make_async_remote_copy(
                    src_ref=sblk.at[b],
                    dst_ref=rbuf.at[b],
                    send_sem=ssem.at[b],
                    recv_sem=rsem.at[b],
                    device_id=(1 - g, y, z),
                    device_id_type=pl.DeviceIdType.MESH,
                ).start()

            @pl.when(i == _NB + b)
            def _():
                pltpu.make_async_remote_copy(
                    src_ref=sblk.at[b],
                    dst_ref=rbuf.at[b],
                    send_sem=ssem.at[b],
                    recv_sem=rsem.at[b],
                    device_id=(1 - g, y, z),
                    device_id_type=pl.DeviceIdType.MESH,
                ).wait_recv()
                o_ref[...] = (acc[...] + rbuf[b].astype(jnp.float32)).astype(
                    jnp.bfloat16
                )

        @pl.when(i == 2 * _NB - 1)
        def _():
            for b in range(_NB):
                pltpu.make_async_remote_copy(
                    src_ref=sblk.at[b],
                    dst_ref=rbuf.at[b],
                    send_sem=ssem.at[b],
                    recv_sem=rsem.at[b],
                    device_id=(1 - g, y, z),
                    device_id_type=pl.DeviceIdType.MESH,
                ).wait_send()


def _matmul_reduce(scalars, x, dy):
    grid = (2 * _NB, K // _BK)
    grid_spec = pltpu.PrefetchScalarGridSpec(
        num_scalar_prefetch=1,
        grid=grid,
        in_specs=[
            pl.BlockSpec(
                (_BK, _BM),
                lambda i, k, s: (
                    k,
                    jnp.where(
                        i < _NB,
                        (1 - s[0]) * _NB + i,
                        s[0] * _NB + (i - _NB),
                    ),
                ),
            ),
            pl.BlockSpec((_BK, SW), lambda i, k, s: (k, s[1])),
        ],
        out_specs=pl.BlockSpec(
            (_BM, SW),
            lambda i, k, s: (jnp.where(i < _NB, _NB, i - _NB), 0),
        ),
        scratch_shapes=[
            pltpu.VMEM((_BM, SW), jnp.float32),
            pltpu.VMEM((_NB, _BM, SW), jnp.bfloat16),
            pltpu.VMEM((_NB, _BM, SW), jnp.bfloat16),
            pltpu.SemaphoreType.DMA((_NB,)),
            pltpu.SemaphoreType.DMA((_NB,)),
        ],
    )
    return pl.pallas_call(
        _mm_body,
        grid_spec=grid_spec,
        out_shape=jax.ShapeDtypeStruct((HALF + _BM, SW), jnp.bfloat16),
        compiler_params=pltpu.CompilerParams(
            dimension_semantics=("arbitrary", "arbitrary"),
        ),
    )(scalars, x, dy)


def _comm_body(f_ref, out_ref, copy_sem, a_send, a_recv, b_send, b_recv):
    g = lax.axis_index("x")
    y = lax.axis_index("y")
    z = lax.axis_index("z")
    p = _h1_pos(y, z)
    q = _h2_pos(y, z)
    ay, az = _h1_yz((p + 1) % NDEV)
    by, bz = _h2_yz((q + 1) % NDEV)

    cp = pltpu.make_async_copy(
        f_ref.at[pl.ds(0, HALF), :], out_ref.at[:, pl.ds(p * SW, SW)], copy_sem
    )
    cp.start()
    cp.wait()

    for h in range(NDEV - 1):
        oa = (p - h) % NDEV
        qb = (q - h) % NDEV
        oby, obz = _h2_yz(qb)
        ob = _h1_pos(oby, obz)

        rdma_a = pltpu.make_async_remote_copy(
            src_ref=out_ref.at[:, pl.ds(oa * SW, HW)],
            dst_ref=out_ref.at[:, pl.ds(oa * SW, HW)],
            send_sem=a_send.at[h],
            recv_sem=a_recv.at[h],
            device_id=(g, ay, az),
            device_id_type=pl.DeviceIdType.MESH,
        )
        rdma_b = pltpu.make_async_remote_copy(
            src_ref=out_ref.at[:, pl.ds(ob * SW + HW, HW)],
            dst_ref=out_ref.at[:, pl.ds(ob * SW + HW, HW)],
            send_sem=b_send.at[h],
            recv_sem=b_recv.at[h],
            device_id=(g, by, bz),
            device_id_type=pl.DeviceIdType.MESH,
        )
        rdma_a.start()
        rdma_b.start()
        rdma_a.wait()
        rdma_b.wait()


def _comm(f_strip):
    return pl.pallas_call(
        _comm_body,
        in_specs=[pl.BlockSpec(memory_space=pltpu.MemorySpace.HBM)],
        out_specs=pl.BlockSpec(memory_space=pltpu.MemorySpace.HBM),
        out_shape=jax.ShapeDtypeStruct((HALF, N), jnp.bfloat16),
        scratch_shapes=[
            pltpu.SemaphoreType.DMA,
            pltpu.SemaphoreType.DMA((NDEV - 1,)),
            pltpu.SemaphoreType.DMA((NDEV - 1,)),
            pltpu.SemaphoreType.DMA((NDEV - 1,)),
            pltpu.SemaphoreType.DMA((NDEV - 1,)),
        ],
    )(f_strip)


def kernel(x, dy):
    g = lax.axis_index("x")
    y = lax.axis_index("y")
    z = lax.axis_index("z")
    p = _h1_pos(y, z)

    scalars = jnp.stack([g, p]).astype(jnp.int32)
    f_strip = _matmul_reduce(scalars, x, dy)
    return _comm(f_strip)


# device time: 227230 ns/iter; 5.5091x vs baseline; 1.2126x over previous
import jax
import jax.numpy as jnp
from jax import lax
from jax.experimental import pallas as pl
from jax.experimental.pallas import tpu as pltpu

M = 4096
N = 8192
K = 4096
HALF = M // 2
NDEV = 8
SW = N // NDEV
HW = SW // 2

_BM = 512
_BK = 1024
_NB = HALF // _BM
_NK = K // _BK
_NHOP = NDEV - 1

_LAST_STEP = 2 * _NB * _NK - 1
_SCHED: dict[int, list[tuple[int, int]]] = {}
_TAIL: list[tuple[int, int, int]] = []
for _b in range(_NB):
    for _h in range(_NHOP):
        _s = 8 * _b + 7 + 3 * _h
        if _s < _LAST_STEP:
            _SCHED.setdefault(_s, []).append((_b, _h))
        else:
            _TAIL.append((_s, _b, _h))
_TAIL.sort()


def _h1_pos(y, z):
    return 2 * z + (y + z) % 2


def _h1_yz(p):
    z = p // 2
    y = (p % 2 + z) % 2
    return y, z


def _h2_pos(y, z):
    q0 = z + 2 * (z // 2)
    q1 = jnp.where(z == 0, 7, jnp.where(z == 3, 6, z + 1))
    return jnp.where(y == 0, q0, q1)


def _h2_yz(q):
    y = (q // 2) % 2
    z = ((q + 1) // 2) % 4
    return y, z


def _body(s_ref, x_ref, dy_ref, out_ref, acc, sblk, rbuf, fvm,
          msend, mrecv, csem, a_send, a_recv, b_send, b_recv):
    i = pl.program_id(0)
    k = pl.program_id(1)
    step = i * _NK + k
    g = s_ref[0]
    y = lax.axis_index("y")
    z = lax.axis_index("z")
    p = _h1_pos(y, z)
    q = _h2_pos(y, z)
    ay, az = _h1_yz((p + 1) % NDEV)
    by, bz = _h2_yz((q + 1) % NDEV)

    def a_col(h):
        return ((p - h) % NDEV) * SW

    def b_col(h):
        oby, obz = _h2_yz((q - h) % NDEV)
        return _h1_pos(oby, obz) * SW + HW

    def ring_desc(ring, b, h):
        col = a_col(h) if ring == 0 else b_col(h)
        dst = out_ref.at[pl.ds(b * _BM, _BM), pl.ds(col, HW)]
        if h == 0:
            src = fvm.at[b, :, pl.ds(0 if ring == 0 else HW, HW)]
        else:
            src = out_ref.at[pl.ds(b * _BM, _BM), pl.ds(col, HW)]
        ss, rs = (a_send, a_recv) if ring == 0 else (b_send, b_recv)
        dev = (g, ay, az) if ring == 0 else (g, by, bz)
        return pltpu.make_async_remote_copy(
            src_ref=src, dst_ref=dst,
            send_sem=ss.at[b, h], recv_sem=rs.at[b, h],
            device_id=dev, device_id_type=pl.DeviceIdType.MESH,
        )

    def ring_op(b, h):
        for ring in (0, 1):
            if h > 0:
                col = a_col(h) if ring == 0 else b_col(h)
                dst = out_ref.at[pl.ds(b * _BM, _BM), pl.ds(col, HW)]
                ss, rs = (a_send, a_recv) if ring == 0 else (b_send, b_recv)
                dev = (g, ay, az) if ring == 0 else (g, by, bz)
                pltpu.make_async_remote_copy(
                    src_ref=dst, dst_ref=dst,
                    send_sem=ss.at[b, h - 1], recv_sem=rs.at[b, h - 1],
                    device_id=dev, device_id_type=pl.DeviceIdType.MESH,
                ).wait_recv()
            ring_desc(ring, b, h).start()

    def mirror_desc(b):
        return pltpu.make_async_remote_copy(
            src_ref=sblk.at[b], dst_ref=rbuf.at[b],
            send_sem=msend.at[b], recv_sem=mrecv.at[b],
            device_id=(1 - g, y, z), device_id_type=pl.DeviceIdType.MESH,
        )

    def out_copy(b):
        return pltpu.make_async_copy(
            fvm.at[b], out_ref.at[pl.ds(b * _BM, _BM), pl.ds(p * SW, SW)],
            csem.at[b],
        )

    @pl.when(k == 0)
    def _():
        acc[...] = jnp.zeros_like(acc)

    xb = x_ref[...].astype(jnp.bfloat16)
    db = dy_ref[...].astype(jnp.bfloat16)
    acc[...] += lax.dot_general(
        xb, db, (((0,), (0,)), ((), ())), preferred_element_type=jnp.float32
    )

    @pl.when(k == _NK - 1)
    def _():
        for b in range(_NB):
            @pl.when(i == 2 * b)
            def _():
                sblk[b] = acc[...].astype(jnp.bfloat16)
                mirror_desc(b).start()

            @pl.when(i == 2 * b + 1)
            def _():
                mirror_desc(b).wait_recv()
                fvm[b] = (acc[...] + rbuf[b].astype(jnp.float32)).astype(
                    jnp.bfloat16
                )
                out_copy(b).start()

    for s, ops in sorted(_SCHED.items()):
        @pl.when(step == s)
        def _(ops=ops):
            for b, h in ops:
                ring_op(b, h)

    @pl.when(step == _LAST_STEP)
    def _():
        for _, b, h in _TAIL:
            ring_op(b, h)

        for b in range(_NB):
            for ring in (0, 1):
                col = a_col(_NHOP) if ring == 0 else b_col(_NHOP)
                dst = out_ref.at[pl.ds(b * _BM, _BM), pl.ds(col, HW)]
                ss, rs = (a_send, a_recv) if ring == 0 else (b_send, b_recv)
                dev = (g, ay, az) if ring == 0 else (g, by, bz)
                pltpu.make_async_remote_copy(
                    src_ref=dst, dst_ref=dst,
                    send_sem=ss.at[b, _NHOP - 1], recv_sem=rs.at[b, _NHOP - 1],
                    device_id=dev, device_id_type=pl.DeviceIdType.MESH,
                ).wait_recv()
        for b in range(_NB):
            mirror_desc(b).wait_send()
            out_copy(b).wait()
            for h in range(_NHOP):
                for ring in (0, 1):
                    ring_desc(ring, b, h).wait_send()


def kernel(x, dy):
    g = lax.axis_index("x")
    y = lax.axis_index("y")
    z = lax.axis_index("z")
    p = _h1_pos(y, z)
    scalars = jnp.stack([g, p]).astype(jnp.int32)

    grid_spec = pltpu.PrefetchScalarGridSpec(
        num_scalar_prefetch=1,
        grid=(2 * _NB, _NK),
        in_specs=[
            pl.BlockSpec(
                (_BK, _BM),
                lambda i, k, s: (
                    k,
                    jnp.where(
                        i % 2 == 0,
                        (1 - s[0]) * _NB + i // 2,
                        s[0] * _NB + i // 2,
                    ),
                ),
            ),
            pl.BlockSpec((_BK, SW), lambda i, k, s: (k, s[1])),
        ],
        out_specs=pl.BlockSpec(memory_space=pltpu.MemorySpace.HBM),
        scratch_shapes=[
            pltpu.VMEM((_BM, SW), jnp.float32),
            pltpu.VMEM((_NB, _BM, SW), jnp.bfloat16),
            pltpu.VMEM((_NB, _BM, SW), jnp.bfloat16),
            pltpu.VMEM((_NB, _BM, SW), jnp.bfloat16),
            pltpu.SemaphoreType.DMA((_NB,)),
            pltpu.SemaphoreType.DMA((_NB,)),
            pltpu.SemaphoreType.DMA((_NB,)),
            pltpu.SemaphoreType.DMA((_NB, _NHOP)),
            pltpu.SemaphoreType.DMA((_NB, _NHOP)),
            pltpu.SemaphoreType.DMA((_NB, _NHOP)),
            pltpu.SemaphoreType.DMA((_NB, _NHOP)),
        ],
    )
    return pl.pallas_call(
        _body,
        grid_spec=grid_spec,
        out_shape=jax.ShapeDtypeStruct((HALF, N), jnp.bfloat16),
        compiler_params=pltpu.CompilerParams(
            dimension_semantics=("arbitrary", "arbitrary"),
        ),
    )(scalars, x, dy)
